# baseline (speedup 1.0000x reference)
import functools

import jax
import jax.numpy as jnp
from jax import lax
from jax.experimental import pallas as pl
from jax.experimental.pallas import tpu as pltpu
from jax.experimental.pallas import tpu_sc as plsc

VOCAB = 1000
D = 128
BATCH = 4096
SEQ = 200
B_TOTAL = BATCH * SEQ
NC, NS = 2, 16
NW = NC * NS
CHUNK = 160
SLOTS = B_TOTAL // (NW * CHUNK)
BASE_PER_W = B_TOTAL // NW
NBUF = 4
GA = 2
HEAD = NBUF
MAIN = ((SLOTS - HEAD) // NBUF) * NBUF
TAIL = SLOTS - HEAD - MAIN


@functools.partial(
    pl.kernel,
    out_type=jax.ShapeDtypeStruct((B_TOTAL, D), jnp.float32),
    mesh=plsc.VectorSubcoreMesh(core_axis_name="c", subcore_axis_name="s"),
    scratch_types=(
        [pltpu.VMEM_SHARED((VOCAB, D), jnp.float32),
         pltpu.VMEM((BASE_PER_W,), jnp.int32)]
        + [pltpu.VMEM((CHUNK, D), jnp.float32)] * NBUF
        + [pltpu.SemaphoreType.DMA] * (2 * NBUF)
    ),
)
def _emb_lookup(idx_hbm, table_hbm, out_hbm, table_sh, idx_v, *bufs):
    rows = bufs[:NBUF]
    gsem = bufs[NBUF:2 * NBUF]
    wsem = bufs[2 * NBUF:]
    sid = lax.axis_index("s")
    wid = sid * NC + lax.axis_index("c")
    base = wid * BASE_PER_W

    @pl.when(sid == 0)
    def _():
        pltpu.sync_copy(table_hbm, table_sh)

    pltpu.sync_copy(idx_hbm.at[pl.ds(wid * BASE_PER_W, BASE_PER_W)], idx_v)
    plsc.subcore_barrier()

    def start_gather(j, b):
        pltpu.make_async_copy(
            table_sh.at[idx_v.at[pl.ds(j * CHUNK, CHUNK)]], rows[b], gsem[b]
        ).start()

    def wait_gather(j, b):
        pltpu.make_async_copy(
            table_sh.at[idx_v.at[pl.ds(j * CHUNK, CHUNK)]], rows[b], gsem[b]
        ).wait()

    def start_write(j, b):
        pltpu.make_async_copy(
            rows[b], out_hbm.at[pl.ds(base + j * CHUNK, CHUNK)], wsem[b]).start()

    def wait_write(b):
        pltpu.make_async_copy(
            rows[b], out_hbm.at[pl.ds(base, CHUNK)], wsem[b]).wait()

    def slot(j, b, first):
        wait_gather(j, b)
        start_write(j, b)
        nb = (b + GA) % NBUF
        if not first:
            wait_write(nb)
        start_gather(lax.rem(j + GA, SLOTS), nb)

    for b in range(GA):
        start_gather(b, b)
    for b in range(HEAD):
        slot(b, b, first=b < NBUF - GA)

    def outer(i, carry):
        for b in range(NBUF):
            slot(i * NBUF + b, b, first=False)
        return carry

    lax.fori_loop(1, 1 + MAIN // NBUF, outer, 0)

    for j in range(HEAD + MAIN, SLOTS):
        slot(j, j % NBUF, first=False)

    for k in range(GA):
        wait_gather(k, (SLOTS - GA + k + GA) % NBUF)
    for j in range(SLOTS - (NBUF - GA), SLOTS):
        wait_write(j % NBUF)


def kernel(texts_indices, table):
    idx = texts_indices.reshape(B_TOTAL).astype(jnp.int32)
    out = _emb_lookup(idx, table)
    return out.reshape(BATCH, SEQ, D)


# speedup vs baseline: 1.0122x; 1.0122x over previous
import functools

import jax
import jax.numpy as jnp
from jax import lax
from jax.experimental import pallas as pl
from jax.experimental.pallas import tpu as pltpu
from jax.experimental.pallas import tpu_sc as plsc

VOCAB = 1000
D = 128
BATCH = 4096
SEQ = 200
B_TOTAL = BATCH * SEQ
NC, NS = 2, 16
NW = NC * NS
CHUNK = 128
SLOTS = B_TOTAL // (NW * CHUNK)
BASE_PER_W = B_TOTAL // NW
NBUF = 5
GA = 3
HEAD = NBUF
MAIN = ((SLOTS - HEAD) // NBUF) * NBUF
TAIL = SLOTS - HEAD - MAIN


@functools.partial(
    pl.kernel,
    out_type=jax.ShapeDtypeStruct((B_TOTAL, D), jnp.float32),
    mesh=plsc.VectorSubcoreMesh(core_axis_name="c", subcore_axis_name="s"),
    scratch_types=(
        [pltpu.VMEM_SHARED((VOCAB, D), jnp.float32),
         pltpu.VMEM((BASE_PER_W,), jnp.int32)]
        + [pltpu.VMEM((CHUNK, D), jnp.float32)] * NBUF
        + [pltpu.SemaphoreType.DMA] * (2 * NBUF)
    ),
)
def _emb_lookup(idx_hbm, table_hbm, out_hbm, table_sh, idx_v, *bufs):
    rows = bufs[:NBUF]
    gsem = bufs[NBUF:2 * NBUF]
    wsem = bufs[2 * NBUF:]
    sid = lax.axis_index("s")
    wid = sid * NC + lax.axis_index("c")
    base = wid * BASE_PER_W

    @pl.when(sid == 0)
    def _():
        pltpu.sync_copy(table_hbm, table_sh)

    pltpu.sync_copy(idx_hbm.at[pl.ds(wid * BASE_PER_W, BASE_PER_W)], idx_v)
    plsc.subcore_barrier()

    def start_gather(j, b):
        pltpu.make_async_copy(
            table_sh.at[idx_v.at[pl.ds(j * CHUNK, CHUNK)]], rows[b], gsem[b]
        ).start()

    def wait_gather(j, b):
        pltpu.make_async_copy(
            table_sh.at[idx_v.at[pl.ds(j * CHUNK, CHUNK)]], rows[b], gsem[b]
        ).wait()

    def start_write(j, b):
        pltpu.make_async_copy(
            rows[b], out_hbm.at[pl.ds(base + j * CHUNK, CHUNK)], wsem[b]).start()

    def wait_write(b):
        pltpu.make_async_copy(
            rows[b], out_hbm.at[pl.ds(base, CHUNK)], wsem[b]).wait()

    def slot(j, b, first):
        wait_gather(j, b)
        start_write(j, b)
        nb = (b + GA) % NBUF
        if not first:
            wait_write(nb)
        start_gather(lax.rem(j + GA, SLOTS), nb)

    for b in range(GA):
        start_gather(b, b)
    for b in range(HEAD):
        slot(b, b, first=b < NBUF - GA)

    def outer(i, carry):
        for b in range(NBUF):
            slot(i * NBUF + b, b, first=False)
        return carry

    lax.fori_loop(1, 1 + MAIN // NBUF, outer, 0)

    for j in range(HEAD + MAIN, SLOTS):
        slot(j, j % NBUF, first=False)

    for k in range(GA):
        wait_gather(k, (SLOTS - GA + k + GA) % NBUF)
    for j in range(SLOTS - (NBUF - GA), SLOTS):
        wait_write(j % NBUF)


def kernel(texts_indices, table):
    idx = texts_indices.reshape(B_TOTAL).astype(jnp.int32)
    out = _emb_lookup(idx, table)
    return out.reshape(BATCH, SEQ, D)
